# Initial kernel scaffold; baseline (speedup 1.0000x reference)
import jax
import jax.numpy as jnp
from jax import lax
from jax.experimental import pallas as pl
from jax.experimental.pallas import tpu as pltpu
from jax.experimental.pallas import tpu_sc as plsc

NC = 2
NS = 16
LANE = 16
K = 128

_F32 = jnp.float32
_I32 = jnp.int32


def _mesh():
    return plsc.VectorSubcoreMesh(
        core_axis_name="c", subcore_axis_name="s", num_cores=NC, num_subcores=NS
    )


def _dual_gather(tab0, idx0, tab1, idx1):
    n_out = idx0.shape[0]
    d = tab0.shape[1]
    cpt = n_out // (NS * K)

    def body(t0_h, i0_h, t1_h, i1_h, o0_h, o1_h, idx_v, rows_v, sem):
        c = lax.axis_index("c")
        s = lax.axis_index("s")

        def make(tab_h, ih, oh):
            def chunk(i, carry):
                base = (s * cpt + i) * K
                pltpu.sync_copy(ih.at[pl.ds(base, K)], idx_v)
                pltpu.async_copy(tab_h.at[idx_v], rows_v, sem).wait()
                pltpu.sync_copy(rows_v, oh.at[pl.ds(base, K)])
                return carry
            return chunk

        @pl.when(c == 0)
        def _():
            lax.fori_loop(0, cpt, make(t0_h, i0_h, o0_h), 0)

        @pl.when(c == 1)
        def _():
            lax.fori_loop(0, cpt, make(t1_h, i1_h, o1_h), 0)

    return pl.kernel(
        body,
        out_type=(
            jax.ShapeDtypeStruct((n_out, d), _F32),
            jax.ShapeDtypeStruct((n_out, d), _F32),
        ),
        mesh=_mesh(),
        scratch_types=[
            pltpu.VMEM((K,), _I32),
            pltpu.VMEM((K, d), _F32),
            pltpu.SemaphoreType.DMA,
        ],
    )(tab0, idx0, tab1, idx1)


def _edge_counts(col0, col1, n_nodes):
    e = col0.shape[0]
    nchunk = e // K
    iters = pl.cdiv(nchunk, NS)
    rpt = n_nodes // NS

    def body(c0_h, c1_h, o0_h, o1_h, colv, ones_v, buf, acc):
        c = lax.axis_index("c")
        s = lax.axis_index("s")

        def fill(r, carry):
            ones_v[r, :] = jnp.full((LANE,), 1.0 / LANE, _F32)
            return carry
        lax.fori_loop(0, K, fill, 0)

        def zero(r, carry):
            buf[r, :] = jnp.zeros((LANE,), _F32)
            return carry
        lax.fori_loop(0, rpt, zero, 0)
        pltpu.sync_copy(buf, acc.at[pl.ds(s * rpt, rpt)])
        plsc.subcore_barrier()

        def make(col_h):
            def chunk(i, carry):
                j = s + NS * i

                @pl.when(j < nchunk)
                def _():
                    pltpu.sync_copy(col_h.at[pl.ds(j * K, K)], colv)
                    pltpu.sync_copy(ones_v, acc.at[colv], add=True)
                return carry
            return chunk

        @pl.when(c == 0)
        def _():
            lax.fori_loop(0, iters, make(c0_h), 0)

        @pl.when(c == 1)
        def _():
            lax.fori_loop(0, iters, make(c1_h), 0)

        plsc.subcore_barrier()
        pltpu.sync_copy(acc.at[pl.ds(s * rpt, rpt)], buf)

        @pl.when(c == 0)
        def _():
            pltpu.sync_copy(buf, o0_h.at[pl.ds(s * rpt, rpt)])

        @pl.when(c == 1)
        def _():
            pltpu.sync_copy(buf, o1_h.at[pl.ds(s * rpt, rpt)])

    return pl.kernel(
        body,
        out_type=(
            jax.ShapeDtypeStruct((n_nodes, LANE), _F32),
            jax.ShapeDtypeStruct((n_nodes, LANE), _F32),
        ),
        mesh=_mesh(),
        scratch_types=[
            pltpu.VMEM((K,), _I32),
            pltpu.VMEM((K, LANE), _F32),
            pltpu.VMEM((n_nodes // NS, LANE), _F32),
            pltpu.VMEM_SHARED((n_nodes, LANE), _F32),
        ],
    )(col0, col1)


def _segsum(x2, row, col, n_dst):
    e = row.shape[0]
    nchunk = e // K
    iters = pl.cdiv(nchunk, NS)
    rpt = n_dst // NS
    hw = 128

    def body(x2_h, row_h, col_h, out_h, rowv, row2v, colv, msg, stage, sem, acc):
        c = lax.axis_index("c")
        s = lax.axis_index("s")

        def zero_r(r, carry):
            def zero_l(l, carry2):
                stage[r, pl.ds(l * LANE, LANE)] = jnp.zeros((LANE,), _F32)
                return carry2
            return lax.fori_loop(0, hw // LANE, zero_l, carry)
        lax.fori_loop(0, rpt, zero_r, 0)
        pltpu.sync_copy(stage, acc.at[pl.ds(s * rpt, rpt)])
        plsc.subcore_barrier()

        def chunk(i, carry):
            j = s + NS * i

            @pl.when(j < nchunk)
            def _():
                e0 = j * K
                pltpu.sync_copy(row_h.at[pl.ds(e0, K)], rowv)
                pltpu.sync_copy(col_h.at[pl.ds(e0, K)], colv)

                def adj(k, carry2):
                    row2v[pl.ds(k * LANE, LANE)] = (
                        rowv[pl.ds(k * LANE, LANE)] * 2 + c
                    )
                    return carry2
                lax.fori_loop(0, K // LANE, adj, 0)
                pltpu.async_copy(x2_h.at[row2v], msg, sem).wait()
                pltpu.sync_copy(msg, acc.at[colv], add=True)
            return carry
        lax.fori_loop(0, iters, chunk, 0)
        plsc.subcore_barrier()

        pltpu.sync_copy(acc.at[pl.ds(s * rpt, rpt)], stage)

        @pl.when(c == 0)
        def _():
            pltpu.sync_copy(stage, out_h.at[0, pl.ds(s * rpt, rpt)])

        @pl.when(c == 1)
        def _():
            pltpu.sync_copy(stage, out_h.at[1, pl.ds(s * rpt, rpt)])

    return pl.kernel(
        body,
        out_type=jax.ShapeDtypeStruct((2, n_dst, hw), _F32),
        mesh=_mesh(),
        scratch_types=[
            pltpu.VMEM((K,), _I32),
            pltpu.VMEM((K,), _I32),
            pltpu.VMEM((K,), _I32),
            pltpu.VMEM((K, hw), _F32),
            pltpu.VMEM((n_dst // NS, hw), _F32),
            pltpu.SemaphoreType.DMA,
            pltpu.VMEM_SHARED((n_dst, hw), _F32),
        ],
    )(x2, row, col)


def _sage_update(s2, cnt16, xdst, wlT, wrT, bl, relu):
    n = s2.shape[1]
    h = xdst.shape[1]
    br = 512
    grid = pl.cdiv(n, br)

    def body(slo, shi, c16, xd, wlo, whi, wr, b, o):
        cnt = jnp.sum(c16[...], axis=1, keepdims=True)
        rc = 1.0 / jnp.maximum(cnt, 1.0)
        acc = jnp.dot(slo[...] * rc, wlo[...],
                      preferred_element_type=_F32, precision=lax.Precision.HIGHEST)
        acc = acc + jnp.dot(shi[...] * rc, whi[...],
                            preferred_element_type=_F32, precision=lax.Precision.HIGHEST)
        acc = acc + jnp.dot(xd[...], wr[...],
                            preferred_element_type=_F32, precision=lax.Precision.HIGHEST)
        acc = acc + b[...]
        o[...] = jnp.maximum(acc, 0.0) if relu else acc

    return pl.pallas_call(
        body,
        grid=(grid,),
        in_specs=[
            pl.BlockSpec((br, 128), lambda i: (i, 0)),
            pl.BlockSpec((br, 128), lambda i: (i, 0)),
            pl.BlockSpec((br, LANE), lambda i: (i, 0)),
            pl.BlockSpec((br, h), lambda i: (i, 0)),
            pl.BlockSpec((128, h), lambda i: (0, 0)),
            pl.BlockSpec((128, h), lambda i: (0, 0)),
            pl.BlockSpec((h, h), lambda i: (0, 0)),
            pl.BlockSpec((1, h), lambda i: (0, 0)),
        ],
        out_specs=pl.BlockSpec((br, h), lambda i: (i, 0)),
        out_shape=jax.ShapeDtypeStruct((n, h), _F32),
    )(s2[0], s2[1], cnt16, xdst, wlT[:128], wlT[128:], wrT, bl.reshape(1, h))


def _decoder(zc, za, w1cT, w1aT, b1, w2, b2):
    lp = zc.shape[0]
    h = zc.shape[1]
    br = 512
    grid = lp // br

    def body(zc_r, za_r, wc, wa, b1r, w2r, b2r, o):
        hid = jnp.dot(zc_r[...], wc[...],
                      preferred_element_type=_F32, precision=lax.Precision.HIGHEST)
        hid = hid + jnp.dot(za_r[...], wa[...],
                            preferred_element_type=_F32, precision=lax.Precision.HIGHEST)
        hid = jnp.maximum(hid + b1r[...], 0.0)
        o[0, :] = jnp.sum(hid * w2r[...], axis=1) + b2r[0, 0]

    return pl.pallas_call(
        body,
        grid=(grid,),
        in_specs=[
            pl.BlockSpec((br, h), lambda i: (i, 0)),
            pl.BlockSpec((br, h), lambda i: (i, 0)),
            pl.BlockSpec((h, h), lambda i: (0, 0)),
            pl.BlockSpec((h, h), lambda i: (0, 0)),
            pl.BlockSpec((1, h), lambda i: (0, 0)),
            pl.BlockSpec((1, h), lambda i: (0, 0)),
            pl.BlockSpec((1, 1), lambda i: (0, 0)),
        ],
        out_specs=pl.BlockSpec((1, br), lambda i: (i, 0)),
        out_shape=jax.ShapeDtypeStruct((grid, br), _F32),
    )(zc, za, w1cT, w1aT, b1.reshape(1, h), w2, b2.reshape(1, 1))


def _pad_idx(idx, n):
    return jnp.concatenate([idx.astype(_I32), jnp.zeros((n - idx.shape[0],), _I32)])


def kernel(x_customer, x_article, edge_index_c2a, edge_index_a2c,
           edge_label_index, emb_customer, emb_article,
           wl1_ca, bl1_ca, wr1_ca, wl1_ac, bl1_ac, wr1_ac,
           wl2_ca, bl2_ca, wr2_ca, wl2_ac, bl2_ac, wr2_ac,
           w_dec1, b_dec1, w_dec2, b_dec2):
    n_c = x_customer.shape[0]
    n_a = x_article.shape[0]
    h = emb_customer.shape[1]
    n_lab = edge_label_index.shape[1]

    gran = NS * K
    np_node = pl.cdiv(max(n_c, n_a), gran) * gran
    lp = pl.cdiv(n_lab, gran) * gran

    idx_c = _pad_idx(x_customer[:, 0], np_node)
    idx_a = _pad_idx(x_article[:, 0], np_node)
    xc_p, xa_p = _dual_gather(emb_customer, idx_c, emb_article, idx_a)

    row_a = edge_index_c2a[0].astype(_I32)
    col_a = edge_index_c2a[1].astype(_I32)
    row_c = edge_index_a2c[0].astype(_I32)
    col_c = edge_index_a2c[1].astype(_I32)
    cnt_a, cnt_c = _edge_counts(col_a, col_c, n_a)

    s_a1 = _segsum(xc_p.reshape(-1, 128), row_a, col_a, n_a)
    s_c1 = _segsum(xa_p.reshape(-1, 128), row_c, col_c, n_c)
    a1 = _sage_update(s_a1, cnt_a, xa_p, wl1_ca.T, wr1_ca.T, bl1_ca, relu=True)
    c1 = _sage_update(s_c1, cnt_c, xc_p, wl1_ac.T, wr1_ac.T, bl1_ac, relu=True)

    s_a2 = _segsum(c1.reshape(-1, 128), row_a, col_a, n_a)
    s_c2 = _segsum(a1.reshape(-1, 128), row_c, col_c, n_c)
    a2 = _sage_update(s_a2, cnt_a, a1, wl2_ca.T, wr2_ca.T, bl2_ca, relu=False)
    c2 = _sage_update(s_c2, cnt_c, c1, wl2_ac.T, wr2_ac.T, bl2_ac, relu=False)

    rowp = _pad_idx(edge_label_index[0], lp)
    colp = _pad_idx(edge_label_index[1], lp)
    zc, za = _dual_gather(c2, rowp, a2, colp)
    dec = _decoder(zc, za, w_dec1[:, :h].T, w_dec1[:, h:].T, b_dec1,
                   w_dec2, b_dec2)
    return dec.reshape(-1)[:n_lab]


# speedup vs baseline: 2.7735x; 2.7735x over previous
import jax
import jax.numpy as jnp
from jax import lax
from jax.experimental import pallas as pl
from jax.experimental.pallas import tpu as pltpu
from jax.experimental.pallas import tpu_sc as plsc

NC = 2
NS = 16
LANE = 16
K = 128

_F32 = jnp.float32
_I32 = jnp.int32


def _mesh():
    return plsc.VectorSubcoreMesh(
        core_axis_name="c", subcore_axis_name="s", num_cores=NC, num_subcores=NS
    )


def _dual_gather(tab0, idx0, tab1, idx1):
    n_out = idx0.shape[0]
    d = tab0.shape[1]
    cpt = n_out // (NS * K)

    def body(t0_h, i0_h, t1_h, i1_h, o0_h, o1_h, idx_v, rows_v, sem):
        c = lax.axis_index("c")
        s = lax.axis_index("s")

        def make(tab_h, ih, oh):
            def chunk(i, carry):
                base = (s * cpt + i) * K
                pltpu.sync_copy(ih.at[pl.ds(base, K)], idx_v)
                pltpu.async_copy(tab_h.at[idx_v], rows_v, sem).wait()
                pltpu.sync_copy(rows_v, oh.at[pl.ds(base, K)])
                return carry
            return chunk

        @pl.when(c == 0)
        def _():
            lax.fori_loop(0, cpt, make(t0_h, i0_h, o0_h), 0)

        @pl.when(c == 1)
        def _():
            lax.fori_loop(0, cpt, make(t1_h, i1_h, o1_h), 0)

    return pl.kernel(
        body,
        out_type=(
            jax.ShapeDtypeStruct((n_out, d), _F32),
            jax.ShapeDtypeStruct((n_out, d), _F32),
        ),
        mesh=_mesh(),
        scratch_types=[
            pltpu.VMEM((K,), _I32),
            pltpu.VMEM((K, d), _F32),
            pltpu.SemaphoreType.DMA,
        ],
    )(tab0, idx0, tab1, idx1)


def _edge_counts(col0, col1, n_nodes):
    e = col0.shape[0]
    nchunk = e // K
    iters = pl.cdiv(nchunk, NS)
    rpt = n_nodes // NS
    cpr = rpt // K

    def body(c0_h, c1_h, ones_h, o0_h, o1_h, colv, ones_v, buf, acc):
        c = lax.axis_index("c")
        s = lax.axis_index("s")

        pltpu.sync_copy(ones_h, ones_v)

        def zero_r(r, carry):
            def zero_l(l, carry2):
                buf[r, pl.ds(l * LANE, LANE)] = jnp.zeros((LANE,), _F32)
                return carry2
            return lax.fori_loop(0, 128 // LANE, zero_l, carry)
        lax.fori_loop(0, K, zero_r, 0)

        def zcp(k, carry):
            pltpu.sync_copy(buf, acc.at[pl.ds(s * rpt + k * K, K)])
            return carry
        lax.fori_loop(0, cpr, zcp, 0)
        plsc.subcore_barrier()

        def make(col_h):
            def chunk(i, carry):
                j = s + NS * i

                @pl.when(j < nchunk)
                def _():
                    pltpu.sync_copy(col_h.at[pl.ds(j * K, K)], colv)
                    pltpu.sync_copy(ones_v, acc.at[colv], add=True)
                return carry
            return chunk

        @pl.when(c == 0)
        def _():
            lax.fori_loop(0, iters, make(c0_h), 0)

        @pl.when(c == 1)
        def _():
            lax.fori_loop(0, iters, make(c1_h), 0)

        plsc.subcore_barrier()

        def out_cp(k, carry):
            r0 = s * rpt + k * K
            pltpu.sync_copy(acc.at[pl.ds(r0, K)], buf)

            @pl.when(c == 0)
            def _():
                pltpu.sync_copy(buf, o0_h.at[pl.ds(r0, K)])

            @pl.when(c == 1)
            def _():
                pltpu.sync_copy(buf, o1_h.at[pl.ds(r0, K)])
            return carry
        lax.fori_loop(0, cpr, out_cp, 0)

    return pl.kernel(
        body,
        out_type=(
            jax.ShapeDtypeStruct((n_nodes, 128), _F32),
            jax.ShapeDtypeStruct((n_nodes, 128), _F32),
        ),
        mesh=_mesh(),
        scratch_types=[
            pltpu.VMEM((K,), _I32),
            pltpu.VMEM((K, 128), _F32),
            pltpu.VMEM((K, 128), _F32),
            pltpu.VMEM_SHARED((n_nodes, 128), _F32),
        ],
    )(col0, col1, jnp.ones((K, 128), _F32))


def _segsum(x2, row, col, n_dst):
    e = row.shape[0]
    nchunk = e // K
    iters = pl.cdiv(nchunk, NS)
    rpt = n_dst // NS
    hw = 128

    cpr = rpt // K

    def body(x2_h, row_h, col_h, out_h, rowv, row2v, colv, msg, sem, acc):
        c = lax.axis_index("c")
        s = lax.axis_index("s")

        def zero_r(r, carry):
            def zero_l(l, carry2):
                msg[r, pl.ds(l * LANE, LANE)] = jnp.zeros((LANE,), _F32)
                return carry2
            return lax.fori_loop(0, hw // LANE, zero_l, carry)
        lax.fori_loop(0, K, zero_r, 0)

        def zcp(k, carry):
            pltpu.sync_copy(msg, acc.at[pl.ds(s * rpt + k * K, K)])
            return carry
        lax.fori_loop(0, cpr, zcp, 0)
        plsc.subcore_barrier()

        def chunk(i, carry):
            j = s + NS * i

            @pl.when(j < nchunk)
            def _():
                e0 = j * K
                pltpu.sync_copy(row_h.at[pl.ds(e0, K)], rowv)
                pltpu.sync_copy(col_h.at[pl.ds(e0, K)], colv)

                def adj(k, carry2):
                    row2v[pl.ds(k * LANE, LANE)] = (
                        rowv[pl.ds(k * LANE, LANE)] * 2 + c
                    )
                    return carry2
                lax.fori_loop(0, K // LANE, adj, 0)
                pltpu.async_copy(x2_h.at[row2v], msg, sem).wait()
                pltpu.sync_copy(msg, acc.at[colv], add=True)
            return carry
        lax.fori_loop(0, iters, chunk, 0)
        plsc.subcore_barrier()

        def out_cp(k, carry):
            r0 = s * rpt + k * K
            pltpu.sync_copy(acc.at[pl.ds(r0, K)], msg)

            @pl.when(c == 0)
            def _():
                pltpu.sync_copy(msg, out_h.at[0, pl.ds(r0, K)])

            @pl.when(c == 1)
            def _():
                pltpu.sync_copy(msg, out_h.at[1, pl.ds(r0, K)])
            return carry
        lax.fori_loop(0, cpr, out_cp, 0)

    return pl.kernel(
        body,
        out_type=jax.ShapeDtypeStruct((2, n_dst, hw), _F32),
        mesh=_mesh(),
        scratch_types=[
            pltpu.VMEM((K,), _I32),
            pltpu.VMEM((K,), _I32),
            pltpu.VMEM((K,), _I32),
            pltpu.VMEM((K, hw), _F32),
            pltpu.SemaphoreType.DMA,
            pltpu.VMEM_SHARED((n_dst, hw), _F32),
        ],
    )(x2, row, col)


def _sage_update(s2, cnt16, xdst, wlT, wrT, bl, relu):
    n = s2.shape[1]
    h = xdst.shape[1]
    br = 512
    grid = pl.cdiv(n, br)

    def body(slo, shi, c16, xd, wlo, whi, wr, b, o):
        cnt = c16[...][:, 0:1]
        rc = 1.0 / jnp.maximum(cnt, 1.0)
        acc = jnp.dot(slo[...] * rc, wlo[...],
                      preferred_element_type=_F32, precision=lax.Precision.HIGHEST)
        acc = acc + jnp.dot(shi[...] * rc, whi[...],
                            preferred_element_type=_F32, precision=lax.Precision.HIGHEST)
        acc = acc + jnp.dot(xd[...], wr[...],
                            preferred_element_type=_F32, precision=lax.Precision.HIGHEST)
        acc = acc + b[...]
        o[...] = jnp.maximum(acc, 0.0) if relu else acc

    return pl.pallas_call(
        body,
        grid=(grid,),
        in_specs=[
            pl.BlockSpec((br, 128), lambda i: (i, 0)),
            pl.BlockSpec((br, 128), lambda i: (i, 0)),
            pl.BlockSpec((br, 128), lambda i: (i, 0)),
            pl.BlockSpec((br, h), lambda i: (i, 0)),
            pl.BlockSpec((128, h), lambda i: (0, 0)),
            pl.BlockSpec((128, h), lambda i: (0, 0)),
            pl.BlockSpec((h, h), lambda i: (0, 0)),
            pl.BlockSpec((1, h), lambda i: (0, 0)),
        ],
        out_specs=pl.BlockSpec((br, h), lambda i: (i, 0)),
        out_shape=jax.ShapeDtypeStruct((n, h), _F32),
    )(s2[0], s2[1], cnt16, xdst, wlT[:128], wlT[128:], wrT, bl.reshape(1, h))


def _decoder(zc, za, w1cT, w1aT, b1, w2, b2):
    lp = zc.shape[0]
    h = zc.shape[1]
    br = 512
    grid = lp // br

    def body(zc_r, za_r, wc, wa, b1r, w2r, b2r, o):
        hid = jnp.dot(zc_r[...], wc[...],
                      preferred_element_type=_F32, precision=lax.Precision.HIGHEST)
        hid = hid + jnp.dot(za_r[...], wa[...],
                            preferred_element_type=_F32, precision=lax.Precision.HIGHEST)
        hid = jnp.maximum(hid + b1r[...], 0.0)
        o[...] = jnp.sum(hid * w2r[...], axis=1) + b2r[0, 0]

    return pl.pallas_call(
        body,
        grid=(grid,),
        in_specs=[
            pl.BlockSpec((br, h), lambda i: (i, 0)),
            pl.BlockSpec((br, h), lambda i: (i, 0)),
            pl.BlockSpec((h, h), lambda i: (0, 0)),
            pl.BlockSpec((h, h), lambda i: (0, 0)),
            pl.BlockSpec((1, h), lambda i: (0, 0)),
            pl.BlockSpec((1, h), lambda i: (0, 0)),
            pl.BlockSpec((1, 1), lambda i: (0, 0)),
        ],
        out_specs=pl.BlockSpec((br,), lambda i: (i,)),
        out_shape=jax.ShapeDtypeStruct((lp,), _F32),
    )(zc, za, w1cT, w1aT, b1.reshape(1, h), w2, b2.reshape(1, 1))


def _pad_idx(idx, n):
    return jnp.concatenate([idx.astype(_I32), jnp.zeros((n - idx.shape[0],), _I32)])


def kernel(x_customer, x_article, edge_index_c2a, edge_index_a2c,
           edge_label_index, emb_customer, emb_article,
           wl1_ca, bl1_ca, wr1_ca, wl1_ac, bl1_ac, wr1_ac,
           wl2_ca, bl2_ca, wr2_ca, wl2_ac, bl2_ac, wr2_ac,
           w_dec1, b_dec1, w_dec2, b_dec2):
    n_c = x_customer.shape[0]
    n_a = x_article.shape[0]
    h = emb_customer.shape[1]
    n_lab = edge_label_index.shape[1]

    gran = NS * K
    np_node = pl.cdiv(max(n_c, n_a), gran) * gran
    lp = pl.cdiv(n_lab, gran) * gran

    idx_c = _pad_idx(x_customer[:, 0], np_node)
    idx_a = _pad_idx(x_article[:, 0], np_node)
    xc_p, xa_p = _dual_gather(emb_customer, idx_c, emb_article, idx_a)

    row_a = edge_index_c2a[0].astype(_I32)
    col_a = edge_index_c2a[1].astype(_I32)
    row_c = edge_index_a2c[0].astype(_I32)
    col_c = edge_index_a2c[1].astype(_I32)
    cnt_a, cnt_c = _edge_counts(col_a, col_c, np_node)

    s_a1 = _segsum(xc_p.reshape(-1, 128), row_a, col_a, np_node)
    s_c1 = _segsum(xa_p.reshape(-1, 128), row_c, col_c, np_node)
    a1 = _sage_update(s_a1, cnt_a, xa_p, wl1_ca.T, wr1_ca.T, bl1_ca, relu=True)
    c1 = _sage_update(s_c1, cnt_c, xc_p, wl1_ac.T, wr1_ac.T, bl1_ac, relu=True)

    s_a2 = _segsum(c1.reshape(-1, 128), row_a, col_a, np_node)
    s_c2 = _segsum(a1.reshape(-1, 128), row_c, col_c, np_node)
    a2 = _sage_update(s_a2, cnt_a, a1, wl2_ca.T, wr2_ca.T, bl2_ca, relu=False)
    c2 = _sage_update(s_c2, cnt_c, c1, wl2_ac.T, wr2_ac.T, bl2_ac, relu=False)

    rowp = _pad_idx(edge_label_index[0], lp)
    colp = _pad_idx(edge_label_index[1], lp)
    zc, za = _dual_gather(c2, rowp, a2, colp)
    dec = _decoder(zc, za, w_dec1[:, :h].T, w_dec1[:, h:].T, b_dec1,
                   w_dec2, b_dec2)
    return dec[:n_lab]


# speedup vs baseline: 3.9186x; 1.4129x over previous
import jax
import jax.numpy as jnp
from jax import lax
from jax.experimental import pallas as pl
from jax.experimental.pallas import tpu as pltpu
from jax.experimental.pallas import tpu_sc as plsc

NC = 2
NS = 16
LANE = 16
K = 128

_F32 = jnp.float32
_I32 = jnp.int32


def _mesh():
    return plsc.VectorSubcoreMesh(
        core_axis_name="c", subcore_axis_name="s", num_cores=NC, num_subcores=NS
    )


def _dual_gather(tab0, idx0, tab1, idx1):
    n_out = idx0.shape[0]
    d = tab0.shape[1]
    cpt = n_out // (NS * K)

    pairs = pl.cdiv(cpt, 2)

    def body(t0_h, i0_h, t1_h, i1_h, o0_h, o1_h,
             idx_a, rows_a, idx_b, rows_b, sem_a, sem_b):
        c = lax.axis_index("c")
        s = lax.axis_index("s")

        def make(tab_h, ih, oh):
            def start(i, idxv, rowsv, sem):
                @pl.when(i < cpt)
                def _():
                    base = (s * cpt + i) * K
                    pltpu.sync_copy(ih.at[pl.ds(base, K)], idxv)
                    pltpu.async_copy(tab_h.at[idxv], rowsv, sem)

            def finish(i, idxv, rowsv, sem):
                @pl.when(i < cpt)
                def _():
                    base = (s * cpt + i) * K
                    pltpu.make_async_copy(tab_h.at[idxv], rowsv, sem).wait()
                    pltpu.sync_copy(rowsv, oh.at[pl.ds(base, K)])

            def run():
                start(0, idx_a, rows_a, sem_a)

                def pair(i2, carry):
                    i0 = 2 * i2
                    start(i0 + 1, idx_b, rows_b, sem_b)
                    finish(i0, idx_a, rows_a, sem_a)
                    start(i0 + 2, idx_a, rows_a, sem_a)
                    finish(i0 + 1, idx_b, rows_b, sem_b)
                    return carry
                lax.fori_loop(0, pairs, pair, 0)
            return run

        @pl.when(c == 0)
        def _():
            make(t0_h, i0_h, o0_h)()

        @pl.when(c == 1)
        def _():
            make(t1_h, i1_h, o1_h)()

    return pl.kernel(
        body,
        out_type=(
            jax.ShapeDtypeStruct((n_out, d), _F32),
            jax.ShapeDtypeStruct((n_out, d), _F32),
        ),
        mesh=_mesh(),
        scratch_types=[
            pltpu.VMEM((K,), _I32),
            pltpu.VMEM((K, d), _F32),
            pltpu.VMEM((K,), _I32),
            pltpu.VMEM((K, d), _F32),
            pltpu.SemaphoreType.DMA,
            pltpu.SemaphoreType.DMA,
        ],
    )(tab0, idx0, tab1, idx1)


def _edge_counts(col0, col1, n_nodes):
    e = col0.shape[0]
    nchunk = e // K
    iters = pl.cdiv(nchunk, NS)
    rpt = n_nodes // NS
    cpr = rpt // K

    def body(c0_h, c1_h, ones_h, o0_h, o1_h, colv, colv_b, ones_v, buf,
             sem_a, sem_b, acc):
        c = lax.axis_index("c")
        s = lax.axis_index("s")

        pltpu.sync_copy(ones_h, ones_v)

        def zero_r(r, carry):
            def zero_l(l, carry2):
                buf[r, pl.ds(l * LANE, LANE)] = jnp.zeros((LANE,), _F32)
                return carry2
            return lax.fori_loop(0, 128 // LANE, zero_l, carry)
        lax.fori_loop(0, K, zero_r, 0)

        def zcp(k, carry):
            pltpu.sync_copy(buf, acc.at[pl.ds(s * rpt + k * K, K)])
            return carry
        lax.fori_loop(0, cpr, zcp, 0)
        plsc.subcore_barrier()

        def make(col_h):
            def start(i, cv, sem):
                j = s + NS * i

                @pl.when(j < nchunk)
                def _():
                    pltpu.sync_copy(col_h.at[pl.ds(j * K, K)], cv)
                    pltpu.async_copy(ones_v, acc.at[cv], sem, add=True)

            def finish(i, cv, sem):
                j = s + NS * i

                @pl.when(j < nchunk)
                def _():
                    pltpu.make_async_copy(ones_v, acc.at[cv], sem).wait()

            def run():
                start(0, colv, sem_a)

                def pair(i2, carry):
                    i0 = 2 * i2
                    start(i0 + 1, colv_b, sem_b)
                    finish(i0, colv, sem_a)
                    start(i0 + 2, colv, sem_a)
                    finish(i0 + 1, colv_b, sem_b)
                    return carry
                lax.fori_loop(0, pl.cdiv(iters, 2), pair, 0)
            return run

        @pl.when(c == 0)
        def _():
            make(c0_h)()

        @pl.when(c == 1)
        def _():
            make(c1_h)()

        plsc.subcore_barrier()

        def out_cp(k, carry):
            r0 = s * rpt + k * K
            pltpu.sync_copy(acc.at[pl.ds(r0, K)], buf)

            @pl.when(c == 0)
            def _():
                pltpu.sync_copy(buf, o0_h.at[pl.ds(r0, K)])

            @pl.when(c == 1)
            def _():
                pltpu.sync_copy(buf, o1_h.at[pl.ds(r0, K)])
            return carry
        lax.fori_loop(0, cpr, out_cp, 0)

    return pl.kernel(
        body,
        out_type=(
            jax.ShapeDtypeStruct((n_nodes, 128), _F32),
            jax.ShapeDtypeStruct((n_nodes, 128), _F32),
        ),
        mesh=_mesh(),
        scratch_types=[
            pltpu.VMEM((K,), _I32),
            pltpu.VMEM((K,), _I32),
            pltpu.VMEM((K, 128), _F32),
            pltpu.VMEM((K, 128), _F32),
            pltpu.SemaphoreType.DMA,
            pltpu.SemaphoreType.DMA,
            pltpu.VMEM_SHARED((n_nodes, 128), _F32),
        ],
    )(col0, col1, jnp.ones((K, 128), _F32))


def _segsum(x2, row, col, n_dst):
    e = row.shape[0]
    nchunk = e // K
    iters = pl.cdiv(nchunk, NS)
    rpt = n_dst // NS
    hw = 128

    cpr = rpt // K

    def body(x2_h, row_h, col_h, out_h,
             rowv_a, row2v_a, colv_a, msg_a,
             rowv_b, row2v_b, colv_b, msg_b,
             sem_a, sem_b, acc):
        c = lax.axis_index("c")
        s = lax.axis_index("s")

        def zero_r(r, carry):
            def zero_l(l, carry2):
                msg_a[r, pl.ds(l * LANE, LANE)] = jnp.zeros((LANE,), _F32)
                return carry2
            return lax.fori_loop(0, hw // LANE, zero_l, carry)
        lax.fori_loop(0, K, zero_r, 0)

        def zcp(k, carry):
            pltpu.sync_copy(msg_a, acc.at[pl.ds(s * rpt + k * K, K)])
            return carry
        lax.fori_loop(0, cpr, zcp, 0)
        plsc.subcore_barrier()

        bufs_a = (rowv_a, row2v_a, colv_a, msg_a, sem_a)
        bufs_b = (rowv_b, row2v_b, colv_b, msg_b, sem_b)

        def start(i, bufs):
            rowv, row2v, colv, msg, sem = bufs
            j = s + NS * i

            @pl.when(j < nchunk)
            def _():
                e0 = j * K
                pltpu.sync_copy(row_h.at[pl.ds(e0, K)], rowv)
                pltpu.sync_copy(col_h.at[pl.ds(e0, K)], colv)

                def adj(k, carry2):
                    row2v[pl.ds(k * LANE, LANE)] = (
                        rowv[pl.ds(k * LANE, LANE)] * 2 + c
                    )
                    return carry2
                lax.fori_loop(0, K // LANE, adj, 0)
                pltpu.async_copy(x2_h.at[row2v], msg, sem)

        def finish(i, bufs):
            rowv, row2v, colv, msg, sem = bufs
            j = s + NS * i

            @pl.when(j < nchunk)
            def _():
                pltpu.make_async_copy(x2_h.at[row2v], msg, sem).wait()
                pltpu.sync_copy(msg, acc.at[colv], add=True)

        start(0, bufs_a)

        def pair(i2, carry):
            i0 = 2 * i2
            start(i0 + 1, bufs_b)
            finish(i0, bufs_a)
            start(i0 + 2, bufs_a)
            finish(i0 + 1, bufs_b)
            return carry
        lax.fori_loop(0, pl.cdiv(iters, 2), pair, 0)
        plsc.subcore_barrier()

        def out_cp(k, carry):
            r0 = s * rpt + k * K
            pltpu.sync_copy(acc.at[pl.ds(r0, K)], msg_a)

            @pl.when(c == 0)
            def _():
                pltpu.sync_copy(msg_a, out_h.at[0, pl.ds(r0, K)])

            @pl.when(c == 1)
            def _():
                pltpu.sync_copy(msg_a, out_h.at[1, pl.ds(r0, K)])
            return carry
        lax.fori_loop(0, cpr, out_cp, 0)

    return pl.kernel(
        body,
        out_type=jax.ShapeDtypeStruct((2, n_dst, hw), _F32),
        mesh=_mesh(),
        scratch_types=[
            pltpu.VMEM((K,), _I32),
            pltpu.VMEM((K,), _I32),
            pltpu.VMEM((K,), _I32),
            pltpu.VMEM((K, hw), _F32),
            pltpu.VMEM((K,), _I32),
            pltpu.VMEM((K,), _I32),
            pltpu.VMEM((K,), _I32),
            pltpu.VMEM((K, hw), _F32),
            pltpu.SemaphoreType.DMA,
            pltpu.SemaphoreType.DMA,
            pltpu.VMEM_SHARED((n_dst, hw), _F32),
        ],
    )(x2, row, col)


def _sage_update(s2, cnt16, xdst, wlT, wrT, bl, relu):
    n = s2.shape[1]
    h = xdst.shape[1]
    br = 512
    grid = pl.cdiv(n, br)

    def body(slo, shi, c16, xd, wlo, whi, wr, b, o):
        cnt = c16[...][:, 0:1]
        rc = 1.0 / jnp.maximum(cnt, 1.0)
        acc = jnp.dot(slo[...] * rc, wlo[...],
                      preferred_element_type=_F32, precision=lax.Precision.HIGHEST)
        acc = acc + jnp.dot(shi[...] * rc, whi[...],
                            preferred_element_type=_F32, precision=lax.Precision.HIGHEST)
        acc = acc + jnp.dot(xd[...], wr[...],
                            preferred_element_type=_F32, precision=lax.Precision.HIGHEST)
        acc = acc + b[...]
        o[...] = jnp.maximum(acc, 0.0) if relu else acc

    return pl.pallas_call(
        body,
        grid=(grid,),
        in_specs=[
            pl.BlockSpec((br, 128), lambda i: (i, 0)),
            pl.BlockSpec((br, 128), lambda i: (i, 0)),
            pl.BlockSpec((br, 128), lambda i: (i, 0)),
            pl.BlockSpec((br, h), lambda i: (i, 0)),
            pl.BlockSpec((128, h), lambda i: (0, 0)),
            pl.BlockSpec((128, h), lambda i: (0, 0)),
            pl.BlockSpec((h, h), lambda i: (0, 0)),
            pl.BlockSpec((1, h), lambda i: (0, 0)),
        ],
        out_specs=pl.BlockSpec((br, h), lambda i: (i, 0)),
        out_shape=jax.ShapeDtypeStruct((n, h), _F32),
    )(s2[0], s2[1], cnt16, xdst, wlT[:128], wlT[128:], wrT, bl.reshape(1, h))


def _decoder(zc, za, w1cT, w1aT, b1, w2, b2):
    lp = zc.shape[0]
    h = zc.shape[1]
    br = 512
    grid = lp // br

    def body(zc_r, za_r, wc, wa, b1r, w2r, b2r, o):
        hid = jnp.dot(zc_r[...], wc[...],
                      preferred_element_type=_F32, precision=lax.Precision.HIGHEST)
        hid = hid + jnp.dot(za_r[...], wa[...],
                            preferred_element_type=_F32, precision=lax.Precision.HIGHEST)
        hid = jnp.maximum(hid + b1r[...], 0.0)
        o[...] = jnp.sum(hid * w2r[...], axis=1) + b2r[0, 0]

    return pl.pallas_call(
        body,
        grid=(grid,),
        in_specs=[
            pl.BlockSpec((br, h), lambda i: (i, 0)),
            pl.BlockSpec((br, h), lambda i: (i, 0)),
            pl.BlockSpec((h, h), lambda i: (0, 0)),
            pl.BlockSpec((h, h), lambda i: (0, 0)),
            pl.BlockSpec((1, h), lambda i: (0, 0)),
            pl.BlockSpec((1, h), lambda i: (0, 0)),
            pl.BlockSpec((1, 1), lambda i: (0, 0)),
        ],
        out_specs=pl.BlockSpec((br,), lambda i: (i,)),
        out_shape=jax.ShapeDtypeStruct((lp,), _F32),
    )(zc, za, w1cT, w1aT, b1.reshape(1, h), w2, b2.reshape(1, 1))


def _pad_idx(idx, n):
    return jnp.concatenate([idx.astype(_I32), jnp.zeros((n - idx.shape[0],), _I32)])


def kernel(x_customer, x_article, edge_index_c2a, edge_index_a2c,
           edge_label_index, emb_customer, emb_article,
           wl1_ca, bl1_ca, wr1_ca, wl1_ac, bl1_ac, wr1_ac,
           wl2_ca, bl2_ca, wr2_ca, wl2_ac, bl2_ac, wr2_ac,
           w_dec1, b_dec1, w_dec2, b_dec2):
    n_c = x_customer.shape[0]
    n_a = x_article.shape[0]
    h = emb_customer.shape[1]
    n_lab = edge_label_index.shape[1]

    gran = NS * K
    np_node = pl.cdiv(max(n_c, n_a), gran) * gran
    lp = pl.cdiv(n_lab, gran) * gran

    idx_c = _pad_idx(x_customer[:, 0], np_node)
    idx_a = _pad_idx(x_article[:, 0], np_node)
    xc_p, xa_p = _dual_gather(emb_customer, idx_c, emb_article, idx_a)

    row_a = edge_index_c2a[0].astype(_I32)
    col_a = edge_index_c2a[1].astype(_I32)
    row_c = edge_index_a2c[0].astype(_I32)
    col_c = edge_index_a2c[1].astype(_I32)
    cnt_a, cnt_c = _edge_counts(col_a, col_c, np_node)

    s_a1 = _segsum(xc_p.reshape(-1, 128), row_a, col_a, np_node)
    s_c1 = _segsum(xa_p.reshape(-1, 128), row_c, col_c, np_node)
    a1 = _sage_update(s_a1, cnt_a, xa_p, wl1_ca.T, wr1_ca.T, bl1_ca, relu=True)
    c1 = _sage_update(s_c1, cnt_c, xc_p, wl1_ac.T, wr1_ac.T, bl1_ac, relu=True)

    s_a2 = _segsum(c1.reshape(-1, 128), row_a, col_a, np_node)
    s_c2 = _segsum(a1.reshape(-1, 128), row_c, col_c, np_node)
    a2 = _sage_update(s_a2, cnt_a, a1, wl2_ca.T, wr2_ca.T, bl2_ca, relu=False)
    c2 = _sage_update(s_c2, cnt_c, c1, wl2_ac.T, wr2_ac.T, bl2_ac, relu=False)

    rowp = _pad_idx(edge_label_index[0], lp)
    colp = _pad_idx(edge_label_index[1], lp)
    zc, za = _dual_gather(c2, rowp, a2, colp)
    dec = _decoder(zc, za, w_dec1[:, :h].T, w_dec1[:, h:].T, b_dec1,
                   w_dec2, b_dec2)
    return dec[:n_lab]
